# Initial kernel scaffold; baseline (speedup 1.0000x reference)
import jax
import jax.numpy as jnp
from jax.experimental import pallas as pl


def kernel(features, sample_nodes_0, sample_nodes_1, sample_nodes_2, W_self_0, W_neigh_0, W_self_1, W_neigh_1):
    raise NotImplementedError("write your pallas kernel here")


# speedup vs baseline: 1.0575x; 1.0575x over previous
import functools

import jax
import jax.numpy as jnp
from jax import lax
from jax.experimental import pallas as pl
from jax.experimental.pallas import tpu as pltpu
from jax.experimental.pallas import tpu_sc as plsc

N_NODES = 100000
DIM = 256
FAN = 16
B0 = 1024
B1 = B0 * FAN
B2 = B1 * FAN

NC = 2
NS = 16
NW = NC * NS

W0 = B0 // NW
W1 = B1 // NW
W2P = B1 // NW
CH = 128
PCH = CH // FAN
N1CH = W1 // CH
N2CH = (W2P * FAN) // CH
DB = DIM // 16


def _sc_gather_pool(feat_hbm, sn2_hbm, sn1_hbm, sn0_hbm,
                    sum2_hbm, g1_hbm, g0_hbm,
                    idx2_v, idx1_v, idx0_v, buf_v, acc_v, g0buf_v, sem):
    c = lax.axis_index("c")
    s = lax.axis_index("s")
    wid = s * NC + c

    pltpu.sync_copy(sn2_hbm.at[pl.ds(wid * N2CH, N2CH)], idx2_v)
    pltpu.sync_copy(sn1_hbm.at[pl.ds(wid * N1CH, N1CH)], idx1_v)
    pltpu.sync_copy(sn0_hbm.at[pl.ds(wid * W0, W0)], idx0_v)

    pltpu.async_copy(feat_hbm.at[idx0_v], g0buf_v, sem).wait()
    pltpu.sync_copy(g0buf_v, g0_hbm.at[pl.ds(wid * W0, W0)])

    for j in range(N1CH):
        pltpu.async_copy(feat_hbm.at[idx1_v.at[j]], buf_v, sem).wait()
        pltpu.sync_copy(buf_v, g1_hbm.at[pl.ds(wid * W1 + j * CH, CH)])

    def chunk(j, carry):
        pltpu.async_copy(feat_hbm.at[idx2_v.at[j]], buf_v, sem).wait()

        def row(r, carry2):
            base = r * FAN
            for d in range(DB):
                v = buf_v[base, pl.ds(d * 16, 16)]
                for n in range(1, FAN):
                    v = v + buf_v[base + n, pl.ds(d * 16, 16)]
                acc_v[r, pl.ds(d * 16, 16)] = v
            return carry2

        lax.fori_loop(0, PCH, row, 0)
        pltpu.sync_copy(acc_v, sum2_hbm.at[pl.ds(wid * W2P + j * PCH, PCH)])
        return carry

    lax.fori_loop(0, N2CH, chunk, 0)


_sc_kernel = functools.partial(
    pl.kernel,
    out_type=(
        jax.ShapeDtypeStruct((B1, DIM), jnp.float32),
        jax.ShapeDtypeStruct((B1, DIM), jnp.float32),
        jax.ShapeDtypeStruct((B0, DIM), jnp.float32),
    ),
    mesh=plsc.VectorSubcoreMesh(
        core_axis_name="c", subcore_axis_name="s",
        num_cores=NC, num_subcores=NS),
    scratch_types=(
        pltpu.VMEM((N2CH, CH), jnp.int32),
        pltpu.VMEM((N1CH, CH), jnp.int32),
        pltpu.VMEM((W0,), jnp.int32),
        pltpu.VMEM((CH, DIM), jnp.float32),
        pltpu.VMEM((PCH, DIM), jnp.float32),
        pltpu.VMEM((W0, DIM), jnp.float32),
        pltpu.SemaphoreType.DMA,
    ),
)(_sc_gather_pool)


def _tc_layer0(g1_ref, s2_ref, ws_ref, wn_ref, ph1_ref, pg1_ref):
    g1 = g1_ref[...]
    p2 = s2_ref[...] * (1.0 / FAN)
    a = jnp.maximum(jnp.dot(g1, ws_ref[...], preferred_element_type=jnp.float32), 0.0)
    b = jnp.maximum(jnp.dot(p2, wn_ref[...], preferred_element_type=jnp.float32), 0.0)
    h1 = jnp.concatenate([a, b], axis=1)
    rows = h1.shape[0] // FAN
    ph1_ref[...] = jnp.mean(h1.reshape(rows, FAN, DIM), axis=1)
    pg1_ref[...] = jnp.mean(g1.reshape(rows, FAN, DIM), axis=1)


def _tc_layer1(g0_ref, pg1_ref, ph1_ref, ws0_ref, wn0_ref, ws1_ref, wn1_ref, out_ref):
    a = jnp.maximum(jnp.dot(g0_ref[...], ws0_ref[...], preferred_element_type=jnp.float32), 0.0)
    b = jnp.maximum(jnp.dot(pg1_ref[...], wn0_ref[...], preferred_element_type=jnp.float32), 0.0)
    h0 = jnp.concatenate([a, b], axis=1)
    out_ref[...] = jnp.concatenate([
        jnp.dot(h0, ws1_ref[...], preferred_element_type=jnp.float32),
        jnp.dot(ph1_ref[...], wn1_ref[...], preferred_element_type=jnp.float32),
    ], axis=1)


_TC_ROWS = 1024


def kernel(features, sample_nodes_0, sample_nodes_1, sample_nodes_2,
           W_self_0, W_neigh_0, W_self_1, W_neigh_1):
    sn2 = sample_nodes_2.reshape(B2 // CH, CH)
    sn1 = sample_nodes_1.reshape(B1 // CH, CH)

    sum2, g1, g0 = _sc_kernel(features, sn2, sn1, sample_nodes_0)

    grid = B1 // _TC_ROWS
    ph1, pg1 = pl.pallas_call(
        _tc_layer0,
        grid=(grid,),
        in_specs=[
            pl.BlockSpec((_TC_ROWS, DIM), lambda i: (i, 0)),
            pl.BlockSpec((_TC_ROWS, DIM), lambda i: (i, 0)),
            pl.BlockSpec((DIM, DIM // 2), lambda i: (0, 0)),
            pl.BlockSpec((DIM, DIM // 2), lambda i: (0, 0)),
        ],
        out_specs=[
            pl.BlockSpec((_TC_ROWS // FAN, DIM), lambda i: (i, 0)),
            pl.BlockSpec((_TC_ROWS // FAN, DIM), lambda i: (i, 0)),
        ],
        out_shape=[
            jax.ShapeDtypeStruct((B0, DIM), jnp.float32),
            jax.ShapeDtypeStruct((B0, DIM), jnp.float32),
        ],
    )(g1, sum2, W_self_0, W_neigh_0)

    out = pl.pallas_call(
        _tc_layer1,
        out_shape=jax.ShapeDtypeStruct((B0, DIM), jnp.float32),
    )(g0, pg1, ph1, W_self_0, W_neigh_0, W_self_1, W_neigh_1)
    return out


# speedup vs baseline: 1.5594x; 1.4747x over previous
import functools

import jax
import jax.numpy as jnp
from jax import lax
from jax.experimental import pallas as pl
from jax.experimental.pallas import tpu as pltpu
from jax.experimental.pallas import tpu_sc as plsc

N_NODES = 100000
DIM = 256
FAN = 16
B0 = 1024
B1 = B0 * FAN
B2 = B1 * FAN

NC = 2
NS = 16
NW = NC * NS

W0 = B0 // NW
W1 = B1 // NW
W2P = B1 // NW
CH = 128
PCH = CH // FAN
N1CH = W1 // CH
N2CH = (W2P * FAN) // CH
DB = DIM // 16


def _sc_gather_pool(feat_hbm, sn2_hbm, sn1_hbm, sn0_hbm,
                    sum2_hbm, g1_hbm, g0_hbm,
                    idx2_v, idx1_v, idx0_v, buf_a, buf_b, acc_a, acc_b,
                    g0buf_v, sem_a, sem_b, sem0):
    c = lax.axis_index("c")
    s = lax.axis_index("s")
    wid = s * NC + c

    bufs = (buf_a, buf_b)
    accs = (acc_a, acc_b)
    sems = (sem_a, sem_b)

    pltpu.sync_copy(sn2_hbm.at[pl.ds(wid * N2CH, N2CH)], idx2_v)
    pltpu.sync_copy(sn1_hbm.at[pl.ds(wid * N1CH, N1CH)], idx1_v)
    pltpu.sync_copy(sn0_hbm.at[pl.ds(wid * W0, W0)], idx0_v)

    g0_cp = pltpu.make_async_copy(feat_hbm.at[idx0_v], g0buf_v, sem0)
    g0_cp.start()

    pltpu.make_async_copy(feat_hbm.at[idx1_v.at[0]], bufs[0], sems[0]).start()
    for j in range(N1CH):
        p = j % 2
        if j + 1 < N1CH:
            pltpu.make_async_copy(
                feat_hbm.at[idx1_v.at[j + 1]], bufs[1 - p], sems[1 - p]).start()
        pltpu.make_async_copy(feat_hbm.at[idx1_v.at[j]], bufs[p], sems[p]).wait()
        pltpu.sync_copy(bufs[p], g1_hbm.at[pl.ds(wid * W1 + j * CH, CH)])

    g0_cp.wait()
    pltpu.sync_copy(g0buf_v, g0_hbm.at[pl.ds(wid * W0, W0)])

    def _pool(buf, acc, r, _):
        base = r * FAN
        for d in range(DB):
            v = buf[base, pl.ds(d * 16, 16)]
            for n in range(1, FAN):
                v = v + buf[base + n, pl.ds(d * 16, 16)]
            acc[r, pl.ds(d * 16, 16)] = v
        return _

    def _start2(j, p):
        pltpu.make_async_copy(feat_hbm.at[idx2_v.at[j]], bufs[p], sems[p]).start()

    def _finish2(j, p):
        pltpu.make_async_copy(feat_hbm.at[idx2_v.at[j]], bufs[p], sems[p]).wait()
        lax.fori_loop(0, PCH, functools.partial(_pool, bufs[p], accs[p]), 0)
        pltpu.sync_copy(accs[p], sum2_hbm.at[pl.ds(wid * W2P + j * PCH, PCH)])

    _start2(0, 0)

    def chunk2(jj, carry):
        j0 = 2 * jj
        _start2(j0 + 1, 1)
        _finish2(j0, 0)

        @pl.when(jj + 1 < N2CH // 2)
        def _():
            _start2(j0 + 2, 0)

        _finish2(j0 + 1, 1)
        return carry

    lax.fori_loop(0, N2CH // 2, chunk2, 0)


_sc_kernel = functools.partial(
    pl.kernel,
    out_type=(
        jax.ShapeDtypeStruct((B1, DIM), jnp.float32),
        jax.ShapeDtypeStruct((B1, DIM), jnp.float32),
        jax.ShapeDtypeStruct((B0, DIM), jnp.float32),
    ),
    mesh=plsc.VectorSubcoreMesh(
        core_axis_name="c", subcore_axis_name="s",
        num_cores=NC, num_subcores=NS),
    scratch_types=(
        pltpu.VMEM((N2CH, CH), jnp.int32),
        pltpu.VMEM((N1CH, CH), jnp.int32),
        pltpu.VMEM((W0,), jnp.int32),
        pltpu.VMEM((CH, DIM), jnp.float32),
        pltpu.VMEM((CH, DIM), jnp.float32),
        pltpu.VMEM((PCH, DIM), jnp.float32),
        pltpu.VMEM((PCH, DIM), jnp.float32),
        pltpu.VMEM((W0, DIM), jnp.float32),
        pltpu.SemaphoreType.DMA,
        pltpu.SemaphoreType.DMA,
        pltpu.SemaphoreType.DMA,
    ),
)(_sc_gather_pool)


def _tc_layer0(g1_ref, s2_ref, ws_ref, wn_ref, ph1_ref, pg1_ref):
    g1 = g1_ref[...]
    p2 = s2_ref[...] * (1.0 / FAN)
    a = jnp.maximum(jnp.dot(g1, ws_ref[...], preferred_element_type=jnp.float32), 0.0)
    b = jnp.maximum(jnp.dot(p2, wn_ref[...], preferred_element_type=jnp.float32), 0.0)
    h1 = jnp.concatenate([a, b], axis=1)
    rows = h1.shape[0] // FAN
    ph1_ref[...] = jnp.mean(h1.reshape(rows, FAN, DIM), axis=1)
    pg1_ref[...] = jnp.mean(g1.reshape(rows, FAN, DIM), axis=1)


def _tc_layer1(g0_ref, pg1_ref, ph1_ref, ws0_ref, wn0_ref, ws1_ref, wn1_ref, out_ref):
    a = jnp.maximum(jnp.dot(g0_ref[...], ws0_ref[...], preferred_element_type=jnp.float32), 0.0)
    b = jnp.maximum(jnp.dot(pg1_ref[...], wn0_ref[...], preferred_element_type=jnp.float32), 0.0)
    h0 = jnp.concatenate([a, b], axis=1)
    out_ref[...] = jnp.concatenate([
        jnp.dot(h0, ws1_ref[...], preferred_element_type=jnp.float32),
        jnp.dot(ph1_ref[...], wn1_ref[...], preferred_element_type=jnp.float32),
    ], axis=1)


_TC_ROWS = 1024


def kernel(features, sample_nodes_0, sample_nodes_1, sample_nodes_2,
           W_self_0, W_neigh_0, W_self_1, W_neigh_1):
    sn2 = sample_nodes_2.reshape(B2 // CH, CH)
    sn1 = sample_nodes_1.reshape(B1 // CH, CH)

    sum2, g1, g0 = _sc_kernel(features, sn2, sn1, sample_nodes_0)

    grid = B1 // _TC_ROWS
    ph1, pg1 = pl.pallas_call(
        _tc_layer0,
        grid=(grid,),
        in_specs=[
            pl.BlockSpec((_TC_ROWS, DIM), lambda i: (i, 0)),
            pl.BlockSpec((_TC_ROWS, DIM), lambda i: (i, 0)),
            pl.BlockSpec((DIM, DIM // 2), lambda i: (0, 0)),
            pl.BlockSpec((DIM, DIM // 2), lambda i: (0, 0)),
        ],
        out_specs=[
            pl.BlockSpec((_TC_ROWS // FAN, DIM), lambda i: (i, 0)),
            pl.BlockSpec((_TC_ROWS // FAN, DIM), lambda i: (i, 0)),
        ],
        out_shape=[
            jax.ShapeDtypeStruct((B0, DIM), jnp.float32),
            jax.ShapeDtypeStruct((B0, DIM), jnp.float32),
        ],
    )(g1, sum2, W_self_0, W_neigh_0)

    out = pl.pallas_call(
        _tc_layer1,
        out_shape=jax.ShapeDtypeStruct((B0, DIM), jnp.float32),
    )(g0, pg1, ph1, W_self_0, W_neigh_0, W_self_1, W_neigh_1)
    return out
